# baseline (speedup 1.0000x reference)
import functools

import jax
import jax.numpy as jnp
from jax.experimental import pallas as pl
from jax.experimental.pallas import tpu as pltpu
from jax.experimental.pallas import tpu_sc as plsc

_N_EMBED = 8192
_EMBED_DIM = 32
_BT = 512

_NC = 2
_NS = 16
_NW = _NC * _NS
_CHUNK = 128


_RB = 64
_STRIP = 128


def _argmin_body(z_ref, e_ref, esq_ref, idx_ref):
    zsq = jnp.sum(z_ref[...] ** 2, axis=1, keepdims=True)
    zm2 = z_ref[...] * -2.0
    d = jax.lax.dot_general(
        zm2,
        e_ref[...],
        (((1,), (1,)), ((), ())),
        preferred_element_type=jnp.float32,
    )
    n_strip = _N_EMBED // _STRIP
    esq = esq_ref[...]
    lane = (
        jax.lax.broadcasted_iota(jnp.int32, (_RB, _STRIP), 1)
        .astype(jnp.float32)
    )
    big = jnp.float32(3.0e38)
    out = []
    for rb in range(_BT // _RB):
        zsqb = jnp.broadcast_to(
            jax.lax.slice(zsq, (rb * _RB, 0), ((rb + 1) * _RB, 1)),
            (_RB, _STRIP),
        )
        minvec = jnp.full((_RB, _STRIP), big, jnp.float32)
        stripvec = jnp.zeros((_RB, _STRIP), jnp.float32)
        for s in range(n_strip):
            esqb = jnp.broadcast_to(
                jax.lax.slice(esq, (0, s * _STRIP), (1, (s + 1) * _STRIP)),
                (_RB, _STRIP),
            )
            dstrip = jax.lax.slice(
                d,
                (rb * _RB, s * _STRIP),
                ((rb + 1) * _RB, (s + 1) * _STRIP),
            )
            dist = (zsqb + esqb) + dstrip
            mask = dist < minvec
            minvec = jnp.minimum(minvec, dist)
            stripvec = jnp.where(mask, jnp.float32(s), stripvec)
        minrow = jnp.min(minvec, axis=1, keepdims=True)
        cand = jnp.where(
            minvec == minrow, stripvec * jnp.float32(_STRIP) + lane, big
        )
        out.append(jnp.min(cand, axis=1).astype(jnp.int32))
    idx_ref[...] = jnp.concatenate(out, axis=0)[:, None]


def _tc_argmin(z_flat, emb, esq):
    t = z_flat.shape[0]
    return pl.pallas_call(
        _argmin_body,
        grid=(t // _BT,),
        in_specs=[
            pl.BlockSpec((_BT, _EMBED_DIM), lambda i: (i, 0)),
            pl.BlockSpec((_N_EMBED, _EMBED_DIM), lambda i: (0, 0)),
            pl.BlockSpec((1, _N_EMBED), lambda i: (0, 0)),
        ],
        out_specs=pl.BlockSpec((_BT, 1), lambda i: (i, 0)),
        out_shape=jax.ShapeDtypeStruct((t, 1), jnp.int32),
    )(z_flat, emb, esq)


_LANES = 128


def _sc_gather(emb_padded, idx_flat):
    t = idx_flat.shape[0]
    b_per_w = t // _NW
    n_chunk = b_per_w // _CHUNK
    idx2 = idx_flat.reshape(t // _CHUNK, _CHUNK)
    mesh = plsc.VectorSubcoreMesh(core_axis_name="c", subcore_axis_name="s")

    @functools.partial(
        pl.kernel,
        mesh=mesh,
        out_type=jax.ShapeDtypeStruct((t, _LANES), emb_padded.dtype),
        scratch_types=[
            pltpu.VMEM((n_chunk, _CHUNK), jnp.int32),
            pltpu.VMEM((_CHUNK, _LANES), emb_padded.dtype),
            pltpu.VMEM((_CHUNK, _LANES), emb_padded.dtype),
            pltpu.SemaphoreType.DMA,
            pltpu.SemaphoreType.DMA,
        ],
    )
    def gather_kernel(table_hbm, idx_hbm, out_hbm, idx_v, buf0, buf1, s0, s1):
        wid = jax.lax.axis_index("s") * _NC + jax.lax.axis_index("c")
        base = wid * b_per_w
        pltpu.sync_copy(idx_hbm.at[pl.ds(wid * n_chunk, n_chunk)], idx_v)
        bufs = (buf0, buf1)
        sems = (s0, s1)
        copies = [None] * n_chunk
        copies[0] = pltpu.async_copy(table_hbm.at[idx_v.at[0]], buf0, s0)
        for c in range(n_chunk):
            if c + 1 < n_chunk:
                copies[c + 1] = pltpu.async_copy(
                    table_hbm.at[idx_v.at[c + 1]],
                    bufs[(c + 1) % 2],
                    sems[(c + 1) % 2],
                )
            copies[c].wait()
            pltpu.sync_copy(
                bufs[c % 2], out_hbm.at[pl.ds(base + c * _CHUNK, _CHUNK)]
            )

    return gather_kernel(emb_padded, idx2)


_N_PIPE = 4


def kernel(z, embeddings):
    z_flat = z.reshape(-1, _EMBED_DIM)
    t = z_flat.shape[0]
    tc = t // _N_PIPE
    esq = jnp.sum(embeddings**2, axis=1)[None, :]
    emb_padded = jnp.pad(embeddings, ((0, 0), (0, _LANES - _EMBED_DIM)))
    idxs = []
    wides = []
    for c in range(_N_PIPE):
        zc = jax.lax.slice(z_flat, (c * tc, 0), ((c + 1) * tc, _EMBED_DIM))
        idxc = _tc_argmin(zc, embeddings, esq).reshape(-1)
        idxs.append(idxc)
        wides.append(_sc_gather(emb_padded, idxc))
    z_q_wide = jnp.concatenate(wides, axis=0)
    z_q = z_q_wide[:, :_EMBED_DIM].reshape(z.shape)
    idx_flat = jnp.concatenate(idxs, axis=0)
    return (z_q, idx_flat.reshape(z.shape[0], -1))


# speedup vs baseline: 1.0153x; 1.0153x over previous
import functools

import jax
import jax.numpy as jnp
from jax.experimental import pallas as pl
from jax.experimental.pallas import tpu as pltpu
from jax.experimental.pallas import tpu_sc as plsc

_N_EMBED = 8192
_EMBED_DIM = 32
_BT = 512

_NC = 2
_NS = 16
_NW = _NC * _NS
_CHUNK = 128


_RB = 64
_STRIP = 128


def _argmin_body(z_ref, e_ref, esq_ref, idx_ref):
    zsq = jnp.sum(z_ref[...] ** 2, axis=1, keepdims=True)
    zm2 = z_ref[...] * -2.0
    d = jax.lax.dot_general(
        zm2,
        e_ref[...],
        (((1,), (1,)), ((), ())),
        preferred_element_type=jnp.float32,
    )
    n_strip = _N_EMBED // _STRIP
    esq = esq_ref[...]
    lane = (
        jax.lax.broadcasted_iota(jnp.int32, (_RB, _STRIP), 1)
        .astype(jnp.float32)
    )
    big = jnp.float32(3.0e38)
    out = []
    for rb in range(_BT // _RB):
        zsqb = jnp.broadcast_to(
            jax.lax.slice(zsq, (rb * _RB, 0), ((rb + 1) * _RB, 1)),
            (_RB, _STRIP),
        )
        minvec = jnp.full((_RB, _STRIP), big, jnp.float32)
        stripvec = jnp.zeros((_RB, _STRIP), jnp.float32)
        for s in range(n_strip):
            esqb = jnp.broadcast_to(
                jax.lax.slice(esq, (0, s * _STRIP), (1, (s + 1) * _STRIP)),
                (_RB, _STRIP),
            )
            dstrip = jax.lax.slice(
                d,
                (rb * _RB, s * _STRIP),
                ((rb + 1) * _RB, (s + 1) * _STRIP),
            )
            dist = (zsqb + esqb) + dstrip
            mask = dist < minvec
            minvec = jnp.minimum(minvec, dist)
            stripvec = jnp.where(mask, jnp.float32(s), stripvec)
        minrow = jnp.min(minvec, axis=1, keepdims=True)
        cand = jnp.where(
            minvec == minrow, stripvec * jnp.float32(_STRIP) + lane, big
        )
        out.append(jnp.min(cand, axis=1).astype(jnp.int32))
    idx_ref[...] = jnp.concatenate(out, axis=0)[:, None]


def _tc_argmin(z_flat, emb, esq):
    t = z_flat.shape[0]
    return pl.pallas_call(
        _argmin_body,
        grid=(t // _BT,),
        in_specs=[
            pl.BlockSpec((_BT, _EMBED_DIM), lambda i: (i, 0)),
            pl.BlockSpec((_N_EMBED, _EMBED_DIM), lambda i: (0, 0)),
            pl.BlockSpec((1, _N_EMBED), lambda i: (0, 0)),
        ],
        out_specs=pl.BlockSpec((_BT, 1), lambda i: (i, 0)),
        out_shape=jax.ShapeDtypeStruct((t, 1), jnp.int32),
    )(z_flat, emb, esq)


_LANES = 128


def _sc_gather(emb_padded, idx_flat):
    t = idx_flat.shape[0]
    b_per_w = t // _NW
    n_chunk = b_per_w // _CHUNK
    idx2 = idx_flat.reshape(t // _CHUNK, _CHUNK)
    mesh = plsc.VectorSubcoreMesh(core_axis_name="c", subcore_axis_name="s")

    @functools.partial(
        pl.kernel,
        mesh=mesh,
        out_type=jax.ShapeDtypeStruct((t, _LANES), emb_padded.dtype),
        scratch_types=[
            pltpu.VMEM((n_chunk, _CHUNK), jnp.int32),
            pltpu.VMEM((_CHUNK, _LANES), emb_padded.dtype),
            pltpu.VMEM((_CHUNK, _LANES), emb_padded.dtype),
            pltpu.SemaphoreType.DMA,
            pltpu.SemaphoreType.DMA,
        ],
    )
    def gather_kernel(table_hbm, idx_hbm, out_hbm, idx_v, buf0, buf1, s0, s1):
        wid = jax.lax.axis_index("s") * _NC + jax.lax.axis_index("c")
        base = wid * b_per_w
        pltpu.sync_copy(idx_hbm.at[pl.ds(wid * n_chunk, n_chunk)], idx_v)
        bufs = (buf0, buf1)
        sems = (s0, s1)
        copies = [None] * n_chunk
        copies[0] = pltpu.async_copy(table_hbm.at[idx_v.at[0]], buf0, s0)
        for c in range(n_chunk):
            if c + 1 < n_chunk:
                copies[c + 1] = pltpu.async_copy(
                    table_hbm.at[idx_v.at[c + 1]],
                    bufs[(c + 1) % 2],
                    sems[(c + 1) % 2],
                )
            copies[c].wait()
            pltpu.sync_copy(
                bufs[c % 2], out_hbm.at[pl.ds(base + c * _CHUNK, _CHUNK)]
            )

    return gather_kernel(emb_padded, idx2)


_N_PIPE = 2


def kernel(z, embeddings):
    z_flat = z.reshape(-1, _EMBED_DIM)
    t = z_flat.shape[0]
    tc = t // _N_PIPE
    esq = jnp.sum(embeddings**2, axis=1)[None, :]
    emb_padded = jnp.pad(embeddings, ((0, 0), (0, _LANES - _EMBED_DIM)))
    idxs = []
    wides = []
    for c in range(_N_PIPE):
        zc = jax.lax.slice(z_flat, (c * tc, 0), ((c + 1) * tc, _EMBED_DIM))
        idxc = _tc_argmin(zc, embeddings, esq).reshape(-1)
        idxs.append(idxc)
        wides.append(_sc_gather(emb_padded, idxc))
    z_q_wide = jnp.concatenate(wides, axis=0)
    z_q = z_q_wide[:, :_EMBED_DIM].reshape(z.shape)
    idx_flat = jnp.concatenate(idxs, axis=0)
    return (z_q, idx_flat.reshape(z.shape[0], -1))


# speedup vs baseline: 1.1947x; 1.1767x over previous
import functools

import jax
import jax.numpy as jnp
from jax.experimental import pallas as pl
from jax.experimental.pallas import tpu as pltpu
from jax.experimental.pallas import tpu_sc as plsc

_N_EMBED = 8192
_EMBED_DIM = 32
_BT = 512

_NC = 2
_NS = 16
_NW = _NC * _NS
_CHUNK = 128


_RB = 64
_STRIP = 128


def _argmin_body(z_ref, e_ref, esq_ref, idx_ref):
    zsq = jnp.sum(z_ref[...] ** 2, axis=1, keepdims=True)
    zm2 = z_ref[...] * -2.0
    d = jax.lax.dot_general(
        zm2,
        e_ref[...],
        (((1,), (1,)), ((), ())),
        preferred_element_type=jnp.float32,
    )
    n_strip = _N_EMBED // _STRIP
    esq = esq_ref[...]
    lane = (
        jax.lax.broadcasted_iota(jnp.int32, (_RB, _STRIP), 1)
        .astype(jnp.float32)
    )
    big = jnp.float32(3.0e38)
    out = []
    for rb in range(_BT // _RB):
        zsqb = jnp.broadcast_to(
            jax.lax.slice(zsq, (rb * _RB, 0), ((rb + 1) * _RB, 1)),
            (_RB, _STRIP),
        )
        minvec = jnp.full((_RB, _STRIP), big, jnp.float32)
        stripvec = jnp.zeros((_RB, _STRIP), jnp.float32)
        for s in range(n_strip):
            esqb = jnp.broadcast_to(
                jax.lax.slice(esq, (0, s * _STRIP), (1, (s + 1) * _STRIP)),
                (_RB, _STRIP),
            )
            dstrip = jax.lax.slice(
                d,
                (rb * _RB, s * _STRIP),
                ((rb + 1) * _RB, (s + 1) * _STRIP),
            )
            dist = (zsqb + esqb) + dstrip
            mask = dist < minvec
            minvec = jnp.minimum(minvec, dist)
            stripvec = jnp.where(mask, jnp.float32(s), stripvec)
        minrow = jnp.min(minvec, axis=1, keepdims=True)
        cand = jnp.where(
            minvec == minrow, stripvec * jnp.float32(_STRIP) + lane, big
        )
        out.append(jnp.min(cand, axis=1).astype(jnp.int32))
    idx_ref[...] = jnp.concatenate(out, axis=0).reshape(1, _BT // 128, 128)


def _tc_argmin(z_flat, emb, esq):
    t = z_flat.shape[0]
    return pl.pallas_call(
        _argmin_body,
        grid=(t // _BT,),
        in_specs=[
            pl.BlockSpec((_BT, _EMBED_DIM), lambda i: (i, 0)),
            pl.BlockSpec((_N_EMBED, _EMBED_DIM), lambda i: (0, 0)),
            pl.BlockSpec((1, _N_EMBED), lambda i: (0, 0)),
        ],
        out_specs=pl.BlockSpec((1, _BT // 128, 128), lambda i: (i, 0, 0)),
        out_shape=jax.ShapeDtypeStruct(
            (t // _BT, _BT // 128, 128), jnp.int32
        ),
    )(z_flat, emb, esq)


_LANES = 128


def _sc_gather(emb_padded, idx2):
    t = idx2.size
    idx2 = idx2.reshape(t // _CHUNK, _CHUNK)
    b_per_w = t // _NW
    n_chunk = b_per_w // _CHUNK
    mesh = plsc.VectorSubcoreMesh(core_axis_name="c", subcore_axis_name="s")

    @functools.partial(
        pl.kernel,
        mesh=mesh,
        out_type=jax.ShapeDtypeStruct((t, _LANES), emb_padded.dtype),
        scratch_types=[
            pltpu.VMEM((n_chunk, _CHUNK), jnp.int32),
            pltpu.VMEM((_CHUNK, _LANES), emb_padded.dtype),
            pltpu.VMEM((_CHUNK, _LANES), emb_padded.dtype),
            pltpu.SemaphoreType.DMA,
            pltpu.SemaphoreType.DMA,
        ],
    )
    def gather_kernel(table_hbm, idx_hbm, out_hbm, idx_v, buf0, buf1, s0, s1):
        wid = jax.lax.axis_index("s") * _NC + jax.lax.axis_index("c")
        base = wid * b_per_w
        pltpu.sync_copy(idx_hbm.at[pl.ds(wid * n_chunk, n_chunk)], idx_v)
        bufs = (buf0, buf1)
        sems = (s0, s1)
        copies = [None] * n_chunk
        copies[0] = pltpu.async_copy(table_hbm.at[idx_v.at[0]], buf0, s0)
        for c in range(n_chunk):
            if c + 1 < n_chunk:
                copies[c + 1] = pltpu.async_copy(
                    table_hbm.at[idx_v.at[c + 1]],
                    bufs[(c + 1) % 2],
                    sems[(c + 1) % 2],
                )
            copies[c].wait()
            pltpu.sync_copy(
                bufs[c % 2], out_hbm.at[pl.ds(base + c * _CHUNK, _CHUNK)]
            )

    return gather_kernel(emb_padded, idx2)


def kernel(z, embeddings):
    z_flat = z.reshape(-1, _EMBED_DIM)
    esq = jnp.sum(embeddings**2, axis=1)[None, :]
    emb_padded = jnp.pad(embeddings, ((0, 0), (0, _LANES - _EMBED_DIM)))
    idx2 = _tc_argmin(z_flat, embeddings, esq)
    z_q_wide = _sc_gather(emb_padded, idx2)
    z_q = z_q_wide[:, :_EMBED_DIM].reshape(z.shape)
    return (z_q, idx2.reshape(z.shape[0], -1))
